# baseline (device time: 16165 ns/iter reference)
import jax
import jax.numpy as jnp
from jax import lax
from jax.experimental import pallas as pl
from jax.experimental.pallas import tpu as pltpu

N_COLS_GLOBAL = 1536


def kernel(x):
    m, n = x.shape

    def body(x_ref, out_ref, acc_ref, send_sem, recv_sem):
        my_x = lax.axis_index("x")
        my_y = lax.axis_index("y")
        peer = (my_x, 1 - my_y)

        barrier_sem = pltpu.get_barrier_semaphore()
        pl.semaphore_signal(
            barrier_sem, inc=1, device_id=peer,
            device_id_type=pl.DeviceIdType.MESH,
        )
        pl.semaphore_wait(barrier_sem, 1)

        ones = jnp.ones((n, 1), jnp.float32)
        acc_ref[0, :, :] = jax.lax.dot_general(
            x_ref[:, :],
            ones,
            dimension_numbers=(((1,), (0,)), ((), ())),
            preferred_element_type=jnp.float32,
        )

        rdma = pltpu.make_async_remote_copy(
            src_ref=acc_ref.at[0],
            dst_ref=acc_ref.at[1],
            send_sem=send_sem,
            recv_sem=recv_sem,
            device_id=peer,
            device_id_type=pl.DeviceIdType.MESH,
        )
        rdma.start()
        rdma.wait()

        out_ref[:, :] = (acc_ref[0, :, :] + acc_ref[1, :, :]) * (
            1.0 / N_COLS_GLOBAL
        )

    return pl.pallas_call(
        body,
        out_shape=jax.ShapeDtypeStruct((m, 1), jnp.float32),
        in_specs=[pl.BlockSpec(memory_space=pltpu.VMEM)],
        out_specs=pl.BlockSpec(memory_space=pltpu.VMEM),
        scratch_shapes=[
            pltpu.VMEM((2, m, 1), jnp.float32),
            pltpu.SemaphoreType.DMA,
            pltpu.SemaphoreType.DMA,
        ],
        compiler_params=pltpu.CompilerParams(collective_id=0),
    )(x)


# device time: 7571 ns/iter; 2.1351x vs baseline; 2.1351x over previous
import jax
import jax.numpy as jnp
from jax import lax
from jax.experimental import pallas as pl
from jax.experimental.pallas import tpu as pltpu

N_COLS_GLOBAL = 1536


def kernel(x):
    m, n = x.shape
    T = m // 128

    def body(x_ref, out_ref, acc_ref, send_sem, recv_sem):
        my_x = lax.axis_index("x")
        my_y = lax.axis_index("y")
        peer = (my_x, 1 - my_y)

        barrier_sem = pltpu.get_barrier_semaphore()
        pl.semaphore_signal(
            barrier_sem, inc=1, device_id=peer,
            device_id_type=pl.DeviceIdType.MESH,
        )

        p = jnp.sum(x_ref[:, :], axis=1, keepdims=True, dtype=jnp.float32)
        acc_ref[0] = p.reshape(T, 128)

        pl.semaphore_wait(barrier_sem, 1)

        rdma = pltpu.make_async_remote_copy(
            src_ref=acc_ref.at[0],
            dst_ref=acc_ref.at[1],
            send_sem=send_sem,
            recv_sem=recv_sem,
            device_id=peer,
            device_id_type=pl.DeviceIdType.MESH,
        )
        rdma.start()
        rdma.wait()

        c = acc_ref[0][:, :] + acc_ref[1][:, :]

        i_row = lax.broadcasted_iota(jnp.int32, (m, T), 0)
        t_col = lax.broadcasted_iota(jnp.int32, (m, T), 1)
        E = (i_row // 128 == t_col).astype(jnp.float32)
        i_row2 = lax.broadcasted_iota(jnp.int32, (m, 128), 0)
        l_col = lax.broadcasted_iota(jnp.int32, (m, 128), 1)
        M = (i_row2 % 128 == l_col).astype(jnp.float32)
        d = jax.lax.dot_general(
            E, c, dimension_numbers=(((1,), (0,)), ((), ())),
            preferred_element_type=jnp.float32,
        )
        out_ref[:, :] = jnp.sum(d * M, axis=1, keepdims=True) * (
            1.0 / N_COLS_GLOBAL
        )

    return pl.pallas_call(
        body,
        out_shape=jax.ShapeDtypeStruct((m, 1), jnp.float32),
        in_specs=[pl.BlockSpec(memory_space=pltpu.VMEM)],
        out_specs=pl.BlockSpec(memory_space=pltpu.VMEM),
        scratch_shapes=[
            pltpu.VMEM((2, T, 128), jnp.float32),
            pltpu.SemaphoreType.DMA,
            pltpu.SemaphoreType.DMA,
        ],
        compiler_params=pltpu.CompilerParams(collective_id=0),
    )(x)


# device time: 6902 ns/iter; 2.3421x vs baseline; 1.0969x over previous
import jax
import jax.numpy as jnp
from jax import lax
from jax.experimental import pallas as pl
from jax.experimental.pallas import tpu as pltpu

N_COLS_GLOBAL = 1536


def kernel(x):
    m, n = x.shape
    T = m // 128

    def body(x_ref, out_ref, acc_ref, send_sem, recv_sem):
        my_x = lax.axis_index("x")
        my_y = lax.axis_index("y")
        peer = (my_x, 1 - my_y)

        barrier_sem = pltpu.get_barrier_semaphore()
        pl.semaphore_signal(
            barrier_sem, inc=1, device_id=peer,
            device_id_type=pl.DeviceIdType.MESH,
        )

        p = jnp.sum(x_ref[:, :], axis=1, keepdims=True, dtype=jnp.float32)
        acc_ref[0] = p.reshape(T, 128)

        pl.semaphore_wait(barrier_sem, 1)

        rdma = pltpu.make_async_remote_copy(
            src_ref=acc_ref.at[0],
            dst_ref=acc_ref.at[1],
            send_sem=send_sem,
            recv_sem=recv_sem,
            device_id=peer,
            device_id_type=pl.DeviceIdType.MESH,
        )
        rdma.start()
        rdma.wait()

        out_ref[:, :] = (acc_ref[0][:, :] + acc_ref[1][:, :]) * (
            1.0 / N_COLS_GLOBAL
        )

    out = pl.pallas_call(
        body,
        out_shape=jax.ShapeDtypeStruct((T, 128), jnp.float32),
        in_specs=[pl.BlockSpec(memory_space=pltpu.VMEM)],
        out_specs=pl.BlockSpec(memory_space=pltpu.VMEM),
        scratch_shapes=[
            pltpu.VMEM((2, T, 128), jnp.float32),
            pltpu.SemaphoreType.DMA,
            pltpu.SemaphoreType.DMA,
        ],
        compiler_params=pltpu.CompilerParams(collective_id=0),
    )(x)
    return out.reshape(m, 1)
